# baseline (speedup 1.0000x reference)
import functools

import jax
import jax.numpy as jnp
from jax import lax
from jax.experimental import pallas as pl
from jax.experimental.pallas import tpu as pltpu
from jax.experimental.pallas import tpu_sc as plsc

N = 5000
NUM_CLASSES = 80
TH = 0.5
L = 16
NS = 16
NC = 2
NW = NS * NC
GPT = -(-NUM_CLASSES // NW)
NPIN = N + L
C2 = 400
CAP = C2 * L
NCP = 128
PAD = -1e30


def _nms_body(x1h, y1h, x2h, y2h, sth, lnh, sgh, outh,
              x1v, y1v, x2v, y2v, arv, alv, stv_, lnv, sgv, ov):
    wid = lax.axis_index("s") * NC + lax.axis_index("c")

    pltpu.sync_copy(x1h, x1v)
    pltpu.sync_copy(y1h, y1v)
    pltpu.sync_copy(x2h, x2v)
    pltpu.sync_copy(y2h, y2v)
    pltpu.sync_copy(sth, stv_)
    pltpu.sync_copy(lnh, lnv)
    pltpu.sync_copy(sgh, sgv)

    lanes = lax.iota(jnp.int32, L)

    def class_body(t, _):
        g = t * NW + wid
        s = stv_[pl.ds(g, L)][0]
        ln = lnv[pl.ds(g, L)][0]
        oc = sgv[pl.ds(g, L)][0]
        nc = (ln + L - 1) // L
        end = s + nc * L

        def init_c(u, _):
            o = s + u * L
            w = jnp.maximum(x2v[pl.ds(o, L)] - x1v[pl.ds(o, L)], 0.0)
            h = jnp.maximum(y2v[pl.ds(o, L)] - y1v[pl.ds(o, L)], 0.0)
            a = w * h
            arv[pl.ds(o, L)] = a
            alv[pl.ds(o, L)] = jnp.where(a > 0.0, 1.0, 0.0)
            return 0

        lax.fori_loop(0, nc, init_c, 0)

        def chunk_body(u, _):
            oi = s + u * L
            x1c = x1v[pl.ds(oi, L)]
            y1c = y1v[pl.ds(oi, L)]
            x2c = x2v[pl.ds(oi, L)]
            y2c = y2v[pl.ds(oi, L)]
            arc = arv[pl.ds(oi, L)]

            def lane_body(li, _):
                gi = oi + li
                a_i = alv[pl.ds(gi, L)][0]

                @pl.when(a_i > 0.0)
                def _():
                    liv = jnp.full((L,), li, jnp.int32)

                    def tk(vec):
                        return vec.at[liv].get(mode="promise_in_bounds")

                    x1i = tk(x1c)
                    y1i = tk(y1c)
                    x2i = tk(x2c)
                    y2i = tk(y2c)
                    ari = tk(arc)

                    def sup_off(o, extra=None):
                        ix1 = jnp.maximum(x1v[pl.ds(o, L)], x1i)
                        iy1 = jnp.maximum(y1v[pl.ds(o, L)], y1i)
                        ix2 = jnp.minimum(x2v[pl.ds(o, L)], x2i)
                        iy2 = jnp.minimum(y2v[pl.ds(o, L)], y2i)
                        inter = (jnp.maximum(ix2 - ix1, 0.0)
                                 * jnp.maximum(iy2 - iy1, 0.0))
                        union = arv[pl.ds(o, L)] + ari - inter
                        sup = inter > union * TH
                        if extra is not None:
                            sup = jnp.logical_and(sup, extra)
                        alv[pl.ds(o, L)] = jnp.where(sup, 0.0,
                                                     alv[pl.ds(o, L)])

                    sup_off(oi, lanes > li)

                    @plsc.parallel_loop(oi + L, end, step=L, unroll=2)
                    def _tail(o):
                        sup_off(o)

                return 0

            lax.fori_loop(0, L, lane_body, 0)

            ov[...] = jnp.where(alv[pl.ds(oi, L)] > 0.0,
                                jnp.full((L,), 1, jnp.int32),
                                jnp.full((L,), 0, jnp.int32))
            pltpu.sync_copy(ov, outh.at[pl.ds((oc + u) * L, L)])
            return 0

        lax.fori_loop(0, nc, chunk_body, 0)
        return 0

    lax.fori_loop(0, GPT, class_body, 0)


_nms_sc = functools.partial(
    pl.kernel,
    out_type=jax.ShapeDtypeStruct((CAP,), jnp.int32),
    mesh=plsc.VectorSubcoreMesh(core_axis_name="c", subcore_axis_name="s",
                                num_cores=NC, num_subcores=NS),
    scratch_types=[
        pltpu.VMEM((NPIN,), jnp.float32),
        pltpu.VMEM((NPIN,), jnp.float32),
        pltpu.VMEM((NPIN,), jnp.float32),
        pltpu.VMEM((NPIN,), jnp.float32),
        pltpu.VMEM((NPIN + L,), jnp.float32),
        pltpu.VMEM((NPIN + L,), jnp.float32),
        pltpu.VMEM((NCP,), jnp.int32),
        pltpu.VMEM((NCP,), jnp.int32),
        pltpu.VMEM((NCP,), jnp.int32),
        pltpu.VMEM((L,), jnp.int32),
    ],
)(_nms_body)


def kernel(boxes, scores, class_ids):
    max_c = boxes.max()
    cls = class_ids.astype(jnp.int32)
    offs = class_ids.astype(boxes.dtype) * (max_c + 1.0)
    b = boxes + offs[:, None]
    order = jnp.argsort(-scores)
    cls_r = cls[order]
    perm = jnp.argsort(cls_r, stable=True)
    ord2 = order[perm]
    grouped = b[ord2]

    cnt = jnp.bincount(cls, length=NUM_CLASSES).astype(jnp.int32)
    nch = (cnt + L - 1) // L
    seg_c = jnp.concatenate([jnp.zeros((1,), jnp.int32),
                             jnp.cumsum(nch)[:-1].astype(jnp.int32)])
    unp_start = jnp.concatenate([jnp.zeros((1,), jnp.int32),
                                 jnp.cumsum(cnt)[:-1].astype(jnp.int32)])

    padc = jnp.full((L,), PAD, jnp.float32)
    x1 = jnp.concatenate([grouped[:, 0], padc])
    y1 = jnp.concatenate([grouped[:, 1], padc])
    x2 = jnp.concatenate([grouped[:, 2], padc])
    y2 = jnp.concatenate([grouped[:, 3], padc])
    padi = jnp.zeros((NCP - NUM_CLASSES,), jnp.int32)
    stp = jnp.concatenate([unp_start, padi])
    lnp = jnp.concatenate([cnt, padi])
    sgp = jnp.concatenate([seg_c, padi])

    keep01 = _nms_sc(x1, y1, x2, y2, stp, lnp, sgp)

    cls_s = cls_r[perm]
    delta = seg_c * L - unp_start
    pp = delta[cls_s] + jnp.arange(N, dtype=jnp.int32)
    kg = keep01[pp]
    vals = jnp.where(kg > 0, ord2, -1)
    return jnp.full((N,), -1, jnp.int32).at[perm].set(vals)


# speedup vs baseline: 1.1008x; 1.1008x over previous
import functools

import jax
import jax.numpy as jnp
from jax import lax
from jax.experimental import pallas as pl
from jax.experimental.pallas import tpu as pltpu
from jax.experimental.pallas import tpu_sc as plsc

N = 5000
NUM_CLASSES = 80
TH = 0.5
L = 16
NS = 16
NC = 1
NW = NS * NC
GPT = -(-NUM_CLASSES // NW)
NPIN = N + L
C2 = 400
CAP = C2 * L
NCP = 128
PAD = -1e30


def _nms_body(x1h, y1h, x2h, y2h, sth, lnh, sgh, outh,
              x1v, y1v, x2v, y2v, arv, alv, stv_, lnv, sgv, ov):
    wid = lax.axis_index("s") * NC + lax.axis_index("c")

    pltpu.sync_copy(x1h, x1v)
    pltpu.sync_copy(y1h, y1v)
    pltpu.sync_copy(x2h, x2v)
    pltpu.sync_copy(y2h, y2v)
    pltpu.sync_copy(sth, stv_)
    pltpu.sync_copy(lnh, lnv)
    pltpu.sync_copy(sgh, sgv)

    lanes = lax.iota(jnp.int32, L)

    def class_body(t, _):
        g = t * NW + wid
        s = stv_[pl.ds(g, L)][0]
        ln = lnv[pl.ds(g, L)][0]
        oc = sgv[pl.ds(g, L)][0]
        nc = (ln + L - 1) // L
        end = s + nc * L

        def init_c(u, _):
            o = s + u * L
            w = jnp.maximum(x2v[pl.ds(o, L)] - x1v[pl.ds(o, L)], 0.0)
            h = jnp.maximum(y2v[pl.ds(o, L)] - y1v[pl.ds(o, L)], 0.0)
            a = w * h
            arv[pl.ds(o, L)] = a
            alv[pl.ds(o, L)] = jnp.where(a > 0.0, 1.0, 0.0)
            return 0

        lax.fori_loop(0, nc, init_c, 0)

        def chunk_body(u, _):
            oi = s + u * L
            x1c = x1v[pl.ds(oi, L)]
            y1c = y1v[pl.ds(oi, L)]
            x2c = x2v[pl.ds(oi, L)]
            y2c = y2v[pl.ds(oi, L)]
            arc = arv[pl.ds(oi, L)]

            def lane_body(li, _):
                gi = oi + li
                a_i = alv[pl.ds(gi, L)][0]

                @pl.when(a_i > 0.0)
                def _():
                    liv = jnp.full((L,), li, jnp.int32)

                    def tk(vec):
                        return vec.at[liv].get(mode="promise_in_bounds")

                    x1i = tk(x1c)
                    y1i = tk(y1c)
                    x2i = tk(x2c)
                    y2i = tk(y2c)
                    ari = tk(arc)

                    def sup_off(o, extra=None):
                        ix1 = jnp.maximum(x1v[pl.ds(o, L)], x1i)
                        iy1 = jnp.maximum(y1v[pl.ds(o, L)], y1i)
                        ix2 = jnp.minimum(x2v[pl.ds(o, L)], x2i)
                        iy2 = jnp.minimum(y2v[pl.ds(o, L)], y2i)
                        inter = (jnp.maximum(ix2 - ix1, 0.0)
                                 * jnp.maximum(iy2 - iy1, 0.0))
                        union = arv[pl.ds(o, L)] + ari - inter
                        sup = inter > union * TH
                        if extra is not None:
                            sup = jnp.logical_and(sup, extra)
                        alv[pl.ds(o, L)] = jnp.where(sup, 0.0,
                                                     alv[pl.ds(o, L)])

                    sup_off(oi, lanes > li)

                    @plsc.parallel_loop(oi + L, end, step=L, unroll=2)
                    def _tail(o):
                        sup_off(o)

                return 0

            lax.fori_loop(0, L, lane_body, 0)

            ov[...] = jnp.where(alv[pl.ds(oi, L)] > 0.0,
                                jnp.full((L,), 1, jnp.int32),
                                jnp.full((L,), 0, jnp.int32))
            pltpu.sync_copy(ov, outh.at[pl.ds((oc + u) * L, L)])
            return 0

        lax.fori_loop(0, nc, chunk_body, 0)
        return 0

    lax.fori_loop(0, GPT, class_body, 0)


_nms_sc = functools.partial(
    pl.kernel,
    out_type=jax.ShapeDtypeStruct((CAP,), jnp.int32),
    mesh=plsc.VectorSubcoreMesh(core_axis_name="c", subcore_axis_name="s",
                                num_cores=NC, num_subcores=NS),
    scratch_types=[
        pltpu.VMEM((NPIN,), jnp.float32),
        pltpu.VMEM((NPIN,), jnp.float32),
        pltpu.VMEM((NPIN,), jnp.float32),
        pltpu.VMEM((NPIN,), jnp.float32),
        pltpu.VMEM((NPIN + L,), jnp.float32),
        pltpu.VMEM((NPIN + L,), jnp.float32),
        pltpu.VMEM((NCP,), jnp.int32),
        pltpu.VMEM((NCP,), jnp.int32),
        pltpu.VMEM((NCP,), jnp.int32),
        pltpu.VMEM((L,), jnp.int32),
    ],
)(_nms_body)


def kernel(boxes, scores, class_ids):
    max_c = boxes.max()
    cls = class_ids.astype(jnp.int32)
    offs = class_ids.astype(boxes.dtype) * (max_c + 1.0)
    b = boxes + offs[:, None]
    order = jnp.argsort(-scores)
    cls_r = cls[order]
    perm = jnp.argsort(cls_r, stable=True)
    ord2 = order[perm]
    grouped = b[ord2]

    cnt = jnp.bincount(cls, length=NUM_CLASSES).astype(jnp.int32)
    nch = (cnt + L - 1) // L
    seg_c = jnp.concatenate([jnp.zeros((1,), jnp.int32),
                             jnp.cumsum(nch)[:-1].astype(jnp.int32)])
    unp_start = jnp.concatenate([jnp.zeros((1,), jnp.int32),
                                 jnp.cumsum(cnt)[:-1].astype(jnp.int32)])

    padc = jnp.full((L,), PAD, jnp.float32)
    x1 = jnp.concatenate([grouped[:, 0], padc])
    y1 = jnp.concatenate([grouped[:, 1], padc])
    x2 = jnp.concatenate([grouped[:, 2], padc])
    y2 = jnp.concatenate([grouped[:, 3], padc])
    padi = jnp.zeros((NCP - NUM_CLASSES,), jnp.int32)
    stp = jnp.concatenate([unp_start, padi])
    lnp = jnp.concatenate([cnt, padi])
    sgp = jnp.concatenate([seg_c, padi])

    keep01 = _nms_sc(x1, y1, x2, y2, stp, lnp, sgp)

    delta = seg_c * L - unp_start
    djump = delta - jnp.concatenate([jnp.zeros((1,), jnp.int32), delta[:-1]])
    pcs = jnp.cumsum(jnp.zeros((N,), jnp.int32).at[unp_start].add(djump))
    pp = pcs + jnp.arange(N, dtype=jnp.int32)
    kg = keep01[pp]
    vals = jnp.where(kg > 0, ord2, -1)
    return jnp.full((N,), -1, jnp.int32).at[perm].set(vals)


# speedup vs baseline: 1.1588x; 1.0527x over previous
import functools

import jax
import jax.numpy as jnp
from jax import lax
from jax.experimental import pallas as pl
from jax.experimental.pallas import tpu as pltpu
from jax.experimental.pallas import tpu_sc as plsc

N = 5000
NUM_CLASSES = 80
TH = 0.5
L = 16
NS = 16
NC = 1
NW = NS * NC
GPT = -(-NUM_CLASSES // NW)
GPAD = 160
NBUF = N + GPAD
C2 = 400
CAP = C2 * L
NCP = 128
PAD = -1e30


def _nms_body(x1h, y1h, x2h, y2h, odh, sth, lnh, sgh, outh,
              x1v, y1v, x2v, y2v, arv, alv, odv, stv_, lnv, sgv, ov, sem):
    wid = lax.axis_index("s") * NC + lax.axis_index("c")

    pltpu.sync_copy(odh, odv)
    pltpu.sync_copy(sth, stv_)
    pltpu.sync_copy(lnh, lnv)
    pltpu.sync_copy(sgh, sgv)

    lanes = lax.iota(jnp.int32, L)

    def class_body(t, _):
        g = t * NW + wid
        s = stv_[pl.ds(g, L)][0]
        ln = lnv[pl.ds(g, L)][0]
        oc = sgv[pl.ds(g, L)][0]
        nc = (ln + L - 1) // L
        end = s + nc * L

        s8 = (s // 8) * 8
        npc = (s - s8 + nc * L + 127) // 128

        def gpiece(p, _):
            o = s8 + p * 128
            idx = odv.at[pl.ds(o, 128)]
            pltpu.async_copy(x1h.at[idx], x1v.at[pl.ds(o, 128)], sem).wait()
            pltpu.async_copy(y1h.at[idx], y1v.at[pl.ds(o, 128)], sem).wait()
            pltpu.async_copy(x2h.at[idx], x2v.at[pl.ds(o, 128)], sem).wait()
            pltpu.async_copy(y2h.at[idx], y2v.at[pl.ds(o, 128)], sem).wait()
            return 0

        lax.fori_loop(0, npc, gpiece, 0)

        def init_c(u, _):
            o = s + u * L
            w = jnp.maximum(x2v[pl.ds(o, L)] - x1v[pl.ds(o, L)], 0.0)
            h = jnp.maximum(y2v[pl.ds(o, L)] - y1v[pl.ds(o, L)], 0.0)
            a = w * h
            arv[pl.ds(o, L)] = a
            alv[pl.ds(o, L)] = jnp.where(
                jnp.logical_and(a > 0.0, lanes < ln - u * L), 1.0, 0.0)
            return 0

        lax.fori_loop(0, nc, init_c, 0)

        def chunk_body(u, _):
            oi = s + u * L
            x1c = x1v[pl.ds(oi, L)]
            y1c = y1v[pl.ds(oi, L)]
            x2c = x2v[pl.ds(oi, L)]
            y2c = y2v[pl.ds(oi, L)]
            arc = arv[pl.ds(oi, L)]

            def lane_body(li, _):
                gi = oi + li
                a_i = alv[pl.ds(gi, L)][0]

                @pl.when(a_i > 0.0)
                def _():
                    liv = jnp.full((L,), li, jnp.int32)

                    def tk(vec):
                        return vec.at[liv].get(mode="promise_in_bounds")

                    x1i = tk(x1c)
                    y1i = tk(y1c)
                    x2i = tk(x2c)
                    y2i = tk(y2c)
                    ari = tk(arc)

                    def sup_off(o, extra=None):
                        ix1 = jnp.maximum(x1v[pl.ds(o, L)], x1i)
                        iy1 = jnp.maximum(y1v[pl.ds(o, L)], y1i)
                        ix2 = jnp.minimum(x2v[pl.ds(o, L)], x2i)
                        iy2 = jnp.minimum(y2v[pl.ds(o, L)], y2i)
                        inter = (jnp.maximum(ix2 - ix1, 0.0)
                                 * jnp.maximum(iy2 - iy1, 0.0))
                        union = arv[pl.ds(o, L)] + ari - inter
                        sup = inter > union * TH
                        if extra is not None:
                            sup = jnp.logical_and(sup, extra)
                        alv[pl.ds(o, L)] = jnp.where(sup, 0.0,
                                                     alv[pl.ds(o, L)])

                    sup_off(oi, lanes > li)

                    @plsc.parallel_loop(oi + L, end, step=L, unroll=2)
                    def _tail(o):
                        sup_off(o)

                return 0

            lax.fori_loop(0, L, lane_body, 0)

            ov[...] = jnp.where(alv[pl.ds(oi, L)] > 0.0,
                                jnp.full((L,), 1, jnp.int32),
                                jnp.full((L,), 0, jnp.int32))
            pltpu.sync_copy(ov, outh.at[pl.ds((oc + u) * L, L)])
            return 0

        lax.fori_loop(0, nc, chunk_body, 0)
        return 0

    lax.fori_loop(0, GPT, class_body, 0)


_nms_sc = functools.partial(
    pl.kernel,
    out_type=jax.ShapeDtypeStruct((CAP,), jnp.int32),
    mesh=plsc.VectorSubcoreMesh(core_axis_name="c", subcore_axis_name="s",
                                num_cores=NC, num_subcores=NS),
    scratch_types=[
        pltpu.VMEM((NBUF,), jnp.float32),
        pltpu.VMEM((NBUF,), jnp.float32),
        pltpu.VMEM((NBUF,), jnp.float32),
        pltpu.VMEM((NBUF,), jnp.float32),
        pltpu.VMEM((NBUF + L,), jnp.float32),
        pltpu.VMEM((NBUF + L,), jnp.float32),
        pltpu.VMEM((NBUF,), jnp.int32),
        pltpu.VMEM((NCP,), jnp.int32),
        pltpu.VMEM((NCP,), jnp.int32),
        pltpu.VMEM((NCP,), jnp.int32),
        pltpu.VMEM((L,), jnp.int32),
        pltpu.SemaphoreType.DMA,
    ],
)(_nms_body)


def kernel(boxes, scores, class_ids):
    max_c = boxes.max()
    cls = class_ids.astype(jnp.int32)
    offs = class_ids.astype(boxes.dtype) * (max_c + 1.0)
    b = boxes + offs[:, None]
    order = jnp.argsort(-scores)
    cls_r = cls[order]
    perm = jnp.argsort(cls_r, stable=True)
    ord2 = order[perm]

    cnt = jnp.bincount(cls, length=NUM_CLASSES).astype(jnp.int32)
    nch = (cnt + L - 1) // L
    seg_c = jnp.concatenate([jnp.zeros((1,), jnp.int32),
                             jnp.cumsum(nch)[:-1].astype(jnp.int32)])
    unp_start = jnp.concatenate([jnp.zeros((1,), jnp.int32),
                                 jnp.cumsum(cnt)[:-1].astype(jnp.int32)])

    odp = jnp.concatenate([ord2, jnp.zeros((GPAD,), jnp.int32)])
    padi = jnp.zeros((NCP - NUM_CLASSES,), jnp.int32)
    stp = jnp.concatenate([unp_start, padi])
    lnp = jnp.concatenate([cnt, padi])
    sgp = jnp.concatenate([seg_c, padi])

    keep01 = _nms_sc(b[:, 0], b[:, 1], b[:, 2], b[:, 3], odp, stp, lnp, sgp)

    delta = seg_c * L - unp_start
    djump = delta - jnp.concatenate([jnp.zeros((1,), jnp.int32), delta[:-1]])
    pcs = jnp.cumsum(jnp.zeros((N,), jnp.int32).at[unp_start].add(djump))
    pp = pcs + jnp.arange(N, dtype=jnp.int32)
    kg = keep01[pp]
    vals = jnp.where(kg > 0, ord2, -1)
    return jnp.full((N,), -1, jnp.int32).at[perm].set(vals)
